# Initial kernel scaffold; baseline (speedup 1.0000x reference)
import jax
import jax.numpy as jnp
from jax.experimental import pallas as pl


def kernel(unit_type_ids, ability_indices, trait_indices, status_indices, numerical, resistances, defenses, movement_costs, unit_type_table, ability_table, trait_table, status_table, ability_query, trait_query, status_query):
    raise NotImplementedError("write your pallas kernel here")


# speedup vs baseline: 3.1087x; 3.1087x over previous
import functools

import jax
import jax.numpy as jnp
from jax import lax
from jax.experimental import pallas as pl
from jax.experimental.pallas import tpu as pltpu
from jax.experimental.pallas import tpu_sc as plsc

B = 16384
OUT_D = 149
NC = 2
NS = 16
NW = NC * NS
ROWS_PER_W = B // NW
CHUNK = 128
NCHUNK = ROWS_PER_W // CHUNK
NGROUP = CHUNK // 16

COL_UNIT = 0
COL_NUM = 64
COL_AB = 75
COL_TR = 91
COL_ST = 107
COL_RES = 123
COL_DEF = 129
COL_MOV = 139


def _full(v):
    return jnp.full((16,), v, jnp.int32)


def _prep_exp_table(tab_v, q_v, e_v):
    lanes = lax.iota(jnp.int32, 16)
    s = jnp.zeros((16,), jnp.float32)
    for d in range(16):
        s = s + (plsc.load_gather(tab_v, [lanes, _full(d)])
                 * plsc.load_gather(q_v, [_full(d)]))
    e_v[...] = s
    s0 = plsc.load_gather(e_v, [_full(0)])
    e_v[...] = jnp.exp(s - s0)


def _body(uids, ab_i, tr_i, st_i, num, res, dfs, mov,
          utab, atab, ttab, stab, qa, qt, qs,
          out,
          uids_v, ab_v, tr_v, st_v, num_v, res_v, def_v, mov_v,
          rows_v, out_v, atab_v, ttab_v, stab_v, ea_v, et_v, es_v,
          qa_v, qt_v, qs_v, sem):
    wid = lax.axis_index("s") * NC + lax.axis_index("c")
    base_w = wid * ROWS_PER_W

    pltpu.sync_copy(atab, atab_v.at[pl.ds(0, 14)])
    pltpu.sync_copy(ttab, ttab_v.at[pl.ds(0, 12)])
    pltpu.sync_copy(stab, stab_v.at[pl.ds(0, 4)])
    pltpu.sync_copy(qa, qa_v)
    pltpu.sync_copy(qt, qt_v)
    pltpu.sync_copy(qs, qs_v)
    _prep_exp_table(atab_v, qa_v, ea_v)
    _prep_exp_table(ttab_v, qt_v, et_v)
    _prep_exp_table(stab_v, qs_v, es_v)

    def attend(idx_v, n_l, tab_v, e_v, out_col, rowid):
        idxs = [plsc.load_gather(idx_v, [rowid, _full(l)]) for l in range(n_l)]
        es = [plsc.load_gather(e_v, [ix]) for ix in idxs]
        denom = es[0]
        for e in es[1:]:
            denom = denom + e
        inv = 1.0 / denom
        ws = [e * inv for e in es]
        for d in range(16):
            cold = _full(d)
            acc = ws[0] * plsc.load_gather(tab_v, [idxs[0], cold])
            for l in range(1, n_l):
                acc = acc + ws[l] * plsc.load_gather(tab_v, [idxs[l], cold])
            plsc.store_scatter(out_v, [rowid, _full(out_col + d)], acc)

    def copy_cols(src_v, n_d, out_col, rowid, scale=None):
        for d in range(n_d):
            v = plsc.load_gather(src_v, [rowid, _full(d)])
            if scale is not None:
                v = v * scale
            plsc.store_scatter(out_v, [rowid, _full(out_col + d)], v)

    for c in range(NCHUNK):
        base = base_w + c * CHUNK
        pltpu.sync_copy(uids.at[pl.ds(base, CHUNK)], uids_v)
        pltpu.sync_copy(ab_i.at[pl.ds(base, CHUNK)], ab_v)
        pltpu.sync_copy(tr_i.at[pl.ds(base, CHUNK)], tr_v)
        pltpu.sync_copy(st_i.at[pl.ds(base, CHUNK)], st_v)
        pltpu.sync_copy(num.at[pl.ds(base, CHUNK)], num_v)
        pltpu.sync_copy(res.at[pl.ds(base, CHUNK)], res_v)
        pltpu.sync_copy(dfs.at[pl.ds(base, CHUNK)], def_v)
        pltpu.sync_copy(mov.at[pl.ds(base, CHUNK)], mov_v)
        pltpu.async_copy(utab.at[uids_v], rows_v, sem).wait()

        def group(g, carry):
            rowid = lax.iota(jnp.int32, 16) + g * 16
            copy_cols(rows_v, 64, COL_UNIT, rowid)
            copy_cols(num_v, 11, COL_NUM, rowid)
            attend(ab_v, 4, atab_v, ea_v, COL_AB, rowid)
            attend(tr_v, 3, ttab_v, et_v, COL_TR, rowid)
            attend(st_v, 2, stab_v, es_v, COL_ST, rowid)
            copy_cols(res_v, 6, COL_RES, rowid)
            copy_cols(def_v, 10, COL_DEF, rowid)
            copy_cols(mov_v, 10, COL_MOV, rowid, scale=0.1)
            return carry

        lax.fori_loop(0, NGROUP, group, 0)
        pltpu.sync_copy(out_v, out.at[pl.ds(base, CHUNK)])


def kernel(unit_type_ids, ability_indices, trait_indices, status_indices,
           numerical, resistances, defenses, movement_costs,
           unit_type_table, ability_table, trait_table, status_table,
           ability_query, trait_query, status_query):
    mesh = plsc.VectorSubcoreMesh(core_axis_name="c", subcore_axis_name="s")
    f32 = jnp.float32
    i32 = jnp.int32
    kfn = pl.kernel(
        _body,
        mesh=mesh,
        compiler_params=pltpu.CompilerParams(
            use_tc_tiling_on_sc=False, needs_layout_passes=False),
        out_type=jax.ShapeDtypeStruct((B, OUT_D), f32),
        scratch_types=[
            pltpu.VMEM((CHUNK,), i32),
            pltpu.VMEM((CHUNK, 4), i32),
            pltpu.VMEM((CHUNK, 3), i32),
            pltpu.VMEM((CHUNK, 2), i32),
            pltpu.VMEM((CHUNK, 11), f32),
            pltpu.VMEM((CHUNK, 6), f32),
            pltpu.VMEM((CHUNK, 10), f32),
            pltpu.VMEM((CHUNK, 10), f32),
            pltpu.VMEM((CHUNK, 64), f32),
            pltpu.VMEM((CHUNK, OUT_D), f32),
            pltpu.VMEM((16, 16), f32),
            pltpu.VMEM((16, 16), f32),
            pltpu.VMEM((16, 16), f32),
            pltpu.VMEM((16,), f32),
            pltpu.VMEM((16,), f32),
            pltpu.VMEM((16,), f32),
            pltpu.VMEM((16,), f32),
            pltpu.VMEM((16,), f32),
            pltpu.VMEM((16,), f32),
            pltpu.SemaphoreType.DMA,
        ],
    )
    return kfn(
        unit_type_ids.astype(i32),
        ability_indices.astype(i32),
        trait_indices.astype(i32),
        status_indices.astype(i32),
        numerical, resistances, defenses, movement_costs,
        unit_type_table, ability_table, trait_table, status_table,
        ability_query, trait_query, status_query,
    )


# speedup vs baseline: 3.4752x; 1.1179x over previous
import jax
import jax.numpy as jnp
from jax import lax
from jax.experimental import pallas as pl
from jax.experimental.pallas import tpu as pltpu
from jax.experimental.pallas import tpu_sc as plsc

B = 16384
OUT_D = 149
NC = 2
NS = 16
NW = NC * NS
ROWS_PER_W = B // NW
CHUNK = 128
NCHUNK = ROWS_PER_W // CHUNK
NGROUP = CHUNK // 16

COL_UNIT = 0
COL_NUM = 64
COL_AB = 75
COL_TR = 91
COL_ST = 107
COL_RES = 123
COL_DEF = 129
COL_MOV = 139

OUT_W = CHUNK * OUT_D


def _full(v):
    return jnp.full((16,), v, jnp.int32)


def _prep_exp_table(tab_v, q_v, e_v):
    lanes = lax.iota(jnp.int32, 16)
    s = jnp.zeros((16,), jnp.float32)
    for d in range(16):
        s = s + (plsc.load_gather(tab_v, [lanes, _full(d)])
                 * plsc.load_gather(q_v, [_full(d)]))
    e_v[...] = s
    s0 = plsc.load_gather(e_v, [_full(0)])
    e_v[...] = jnp.exp(s - s0)


def _body(uids, ab_i, tr_i, st_i, num, res, dfs, mov,
          utab, atab, ttab, stab, qa, qt, qs,
          out,
          uids_v, ab_v, tr_v, st_v, num_v, res_v, def_v, mov_v,
          rows_v, out_v, atab_v, ttab_v, stab_v, ea_v, et_v, es_v,
          qa_v, qt_v, qs_v, sem):
    wid = lax.axis_index("s") * NC + lax.axis_index("c")
    base_w = wid * ROWS_PER_W

    pltpu.sync_copy(atab, atab_v.at[pl.ds(0, 14)])
    pltpu.sync_copy(ttab, ttab_v.at[pl.ds(0, 12)])
    pltpu.sync_copy(stab, stab_v.at[pl.ds(0, 4)])
    pltpu.sync_copy(qa, qa_v)
    pltpu.sync_copy(qt, qt_v)
    pltpu.sync_copy(qs, qs_v)
    _prep_exp_table(atab_v, qa_v, ea_v)
    _prep_exp_table(ttab_v, qt_v, et_v)
    _prep_exp_table(stab_v, qs_v, es_v)

    def attend(idx_v, n_l, tab_v, e_v, out_col, rowid, rowoff):
        idxs = [plsc.load_gather(idx_v, [rowid, _full(l)]) for l in range(n_l)]
        es = [plsc.load_gather(e_v, [ix]) for ix in idxs]
        denom = es[0]
        for e in es[1:]:
            denom = denom + e
        inv = 1.0 / denom
        ws = [e * inv for e in es]
        for d in range(16):
            cold = _full(d)
            acc = ws[0] * plsc.load_gather(tab_v, [idxs[0], cold])
            for l in range(1, n_l):
                acc = acc + ws[l] * plsc.load_gather(tab_v, [idxs[l], cold])
            plsc.store_scatter(out_v, [rowoff + _full(out_col + d)], acc)

    for c in range(NCHUNK):
        base = base_w + c * CHUNK
        pltpu.sync_copy(uids.at[pl.ds(base, CHUNK)], uids_v)
        pltpu.sync_copy(ab_i.at[pl.ds(base, CHUNK)], ab_v)
        pltpu.sync_copy(tr_i.at[pl.ds(base, CHUNK)], tr_v)
        pltpu.sync_copy(st_i.at[pl.ds(base, CHUNK)], st_v)
        pltpu.sync_copy(num.at[pl.ds(base * 11, CHUNK * 11)],
                        num_v.at[pl.ds(0, CHUNK * 11)])
        pltpu.sync_copy(res.at[pl.ds(base * 6, CHUNK * 6)],
                        res_v.at[pl.ds(0, CHUNK * 6)])
        pltpu.sync_copy(dfs.at[pl.ds(base * 10, CHUNK * 10)],
                        def_v.at[pl.ds(0, CHUNK * 10)])
        pltpu.sync_copy(mov.at[pl.ds(base * 10, CHUNK * 10)],
                        mov_v.at[pl.ds(0, CHUNK * 10)])
        pltpu.async_copy(utab.at[uids_v], rows_v, sem).wait()

        def group(g, carry):
            rbase = g * 16
            for j in range(16):
                r = rbase + j
                roff = r * OUT_D
                out_v[pl.ds(roff + COL_NUM, 16)] = num_v[pl.ds(r * 11, 16)]
                out_v[pl.ds(roff + COL_RES, 16)] = res_v[pl.ds(r * 6, 16)]
                out_v[pl.ds(roff + COL_DEF, 16)] = def_v[pl.ds(r * 10, 16)]
                out_v[pl.ds(roff + COL_MOV, 16)] = mov_v[pl.ds(r * 10, 16)] * 0.1
            for j in range(16):
                r = rbase + j
                roff = r * OUT_D
                for k in range(4):
                    out_v[pl.ds(roff + k * 16, 16)] = rows_v[r, pl.ds(k * 16, 16)]
            rowid = lax.iota(jnp.int32, 16) + rbase
            rowoff = rowid * OUT_D
            attend(ab_v, 4, atab_v, ea_v, COL_AB, rowid, rowoff)
            attend(tr_v, 3, ttab_v, et_v, COL_TR, rowid, rowoff)
            attend(st_v, 2, stab_v, es_v, COL_ST, rowid, rowoff)
            return carry

        lax.fori_loop(0, NGROUP, group, 0)
        pltpu.sync_copy(out_v.at[pl.ds(0, OUT_W)],
                        out.at[pl.ds(base * OUT_D, OUT_W)])


def kernel(unit_type_ids, ability_indices, trait_indices, status_indices,
           numerical, resistances, defenses, movement_costs,
           unit_type_table, ability_table, trait_table, status_table,
           ability_query, trait_query, status_query):
    mesh = plsc.VectorSubcoreMesh(core_axis_name="c", subcore_axis_name="s")
    f32 = jnp.float32
    i32 = jnp.int32
    kfn = pl.kernel(
        _body,
        mesh=mesh,
        compiler_params=pltpu.CompilerParams(
            use_tc_tiling_on_sc=False, needs_layout_passes=False),
        out_type=jax.ShapeDtypeStruct((B * OUT_D,), f32),
        scratch_types=[
            pltpu.VMEM((CHUNK,), i32),
            pltpu.VMEM((CHUNK, 4), i32),
            pltpu.VMEM((CHUNK, 3), i32),
            pltpu.VMEM((CHUNK, 2), i32),
            pltpu.VMEM((CHUNK * 11 + 16,), f32),
            pltpu.VMEM((CHUNK * 6 + 16,), f32),
            pltpu.VMEM((CHUNK * 10 + 16,), f32),
            pltpu.VMEM((CHUNK * 10 + 16,), f32),
            pltpu.VMEM((CHUNK, 64), f32),
            pltpu.VMEM((OUT_W + 16,), f32),
            pltpu.VMEM((16, 16), f32),
            pltpu.VMEM((16, 16), f32),
            pltpu.VMEM((16, 16), f32),
            pltpu.VMEM((16,), f32),
            pltpu.VMEM((16,), f32),
            pltpu.VMEM((16,), f32),
            pltpu.VMEM((16,), f32),
            pltpu.VMEM((16,), f32),
            pltpu.VMEM((16,), f32),
            pltpu.SemaphoreType.DMA,
        ],
    )
    out_flat = kfn(
        unit_type_ids.astype(i32),
        ability_indices.astype(i32),
        trait_indices.astype(i32),
        status_indices.astype(i32),
        numerical.reshape(-1),
        resistances.reshape(-1),
        defenses.reshape(-1),
        movement_costs.reshape(-1),
        unit_type_table, ability_table, trait_table, status_table,
        ability_query, trait_query, status_query,
    )
    return out_flat.reshape(B, OUT_D)


# speedup vs baseline: 3.7639x; 1.0831x over previous
import jax
import jax.numpy as jnp
from jax import lax
from jax.experimental import pallas as pl
from jax.experimental.pallas import tpu as pltpu
from jax.experimental.pallas import tpu_sc as plsc

B = 16384
OUT_D = 149
NC = 2
NS = 16
NW = NC * NS
ROWS_PER_W = B // NW
CHUNK = 128
NCHUNK = ROWS_PER_W // CHUNK
NGROUP = CHUNK // 16

COL_UNIT = 0
COL_NUM = 64
COL_AB = 75
COL_TR = 91
COL_ST = 107
COL_RES = 123
COL_DEF = 129
COL_MOV = 139

OUT_W = CHUNK * OUT_D


def _full(v):
    return jnp.full((16,), v, jnp.int32)


def _prep_exp_table(tab_v, q_v, e_v):
    lanes = lax.iota(jnp.int32, 16)
    s = jnp.zeros((16,), jnp.float32)
    for d in range(16):
        s = s + (plsc.load_gather(tab_v, [lanes, _full(d)])
                 * plsc.load_gather(q_v, [_full(d)]))
    e_v[...] = s
    s0 = plsc.load_gather(e_v, [_full(0)])
    e_v[...] = jnp.exp(s - s0)


def _body(uids, ab_i, tr_i, st_i, num, res, dfs, mov,
          utab, atab, ttab, stab, qa, qt, qs,
          out,
          uids_v, ab_v, tr_v, st_v, num_v, res_v, def_v, mov_v,
          rows_v, out_v, atab_v, ttab_v, stab_v, ea_v, et_v, es_v,
          qa_v, qt_v, qs_v, sem_in, sem_g, sem_out):
    wid = lax.axis_index("s") * NC + lax.axis_index("c")
    base_w = wid * ROWS_PER_W

    pltpu.sync_copy(atab, atab_v.at[pl.ds(0, 14)])
    pltpu.sync_copy(ttab, ttab_v.at[pl.ds(0, 12)])
    pltpu.sync_copy(stab, stab_v.at[pl.ds(0, 4)])
    pltpu.sync_copy(qa, qa_v)
    pltpu.sync_copy(qt, qt_v)
    pltpu.sync_copy(qs, qs_v)
    _prep_exp_table(atab_v, qa_v, ea_v)
    _prep_exp_table(ttab_v, qt_v, et_v)
    _prep_exp_table(stab_v, qs_v, es_v)

    def stage(c, b):
        base = base_w + c * CHUNK
        mk = pltpu.async_copy
        return [
            mk(uids.at[pl.ds(base, CHUNK)], uids_v.at[b], sem_in.at[b]),
            mk(ab_i.at[pl.ds(base, CHUNK)], ab_v.at[b], sem_in.at[b]),
            mk(tr_i.at[pl.ds(base, CHUNK)], tr_v.at[b], sem_in.at[b]),
            mk(st_i.at[pl.ds(base, CHUNK)], st_v.at[b], sem_in.at[b]),
            mk(num.at[pl.ds(base * 11, CHUNK * 11)],
               num_v.at[b, pl.ds(0, CHUNK * 11)], sem_in.at[b]),
            mk(res.at[pl.ds(base * 6, CHUNK * 6)],
               res_v.at[b, pl.ds(0, CHUNK * 6)], sem_in.at[b]),
            mk(dfs.at[pl.ds(base * 10, CHUNK * 10)],
               def_v.at[b, pl.ds(0, CHUNK * 10)], sem_in.at[b]),
            mk(mov.at[pl.ds(base * 10, CHUNK * 10)],
               mov_v.at[b, pl.ds(0, CHUNK * 10)], sem_in.at[b]),
        ]

    def attend(idx_v, n_l, tab_v, e_v, out_col, rowid, rowoff, outb):
        idxs = [plsc.load_gather(idx_v, [rowid, _full(l)]) for l in range(n_l)]
        es = [plsc.load_gather(e_v, [ix]) for ix in idxs]
        denom = es[0]
        for e in es[1:]:
            denom = denom + e
        inv = 1.0 / denom
        ws = [e * inv for e in es]
        for d in range(16):
            cold = _full(d)
            acc = ws[0] * plsc.load_gather(tab_v, [idxs[0], cold])
            for l in range(1, n_l):
                acc = acc + ws[l] * plsc.load_gather(tab_v, [idxs[l], cold])
            plsc.store_scatter(outb, [rowoff + _full(out_col + d)], acc)

    in_descs = {0: stage(0, 0)}
    g_descs = {}
    out_descs = {}
    for c in range(NCHUNK):
        b = c % 2
        base = base_w + c * CHUNK
        for d in in_descs.pop(c):
            d.wait()
        g_descs[c] = pltpu.async_copy(utab.at[uids_v.at[b]],
                                      rows_v.at[b], sem_g.at[b])
        if c + 1 < NCHUNK:
            in_descs[c + 1] = stage(c + 1, 1 - b)
        if c - 2 >= 0:
            out_descs.pop(c - 2).wait()

        numb, resb, defb, movb = (num_v.at[b], res_v.at[b],
                                  def_v.at[b], mov_v.at[b])
        abb, trb, stb = ab_v.at[b], tr_v.at[b], st_v.at[b]
        outb, rowsb = out_v.at[b], rows_v.at[b]

        def group_ac(g, carry):
            rbase = g * 16
            for j in range(16):
                r = rbase + j
                roff = r * OUT_D
                outb[pl.ds(roff + COL_NUM, 16)] = numb[pl.ds(r * 11, 16)]
                outb[pl.ds(roff + COL_RES, 16)] = resb[pl.ds(r * 6, 16)]
                outb[pl.ds(roff + COL_DEF, 16)] = defb[pl.ds(r * 10, 16)]
                outb[pl.ds(roff + COL_MOV, 16)] = movb[pl.ds(r * 10, 16)] * 0.1
            rowid = lax.iota(jnp.int32, 16) + rbase
            rowoff = rowid * OUT_D
            attend(abb, 4, atab_v, ea_v, COL_AB, rowid, rowoff, outb)
            attend(trb, 3, ttab_v, et_v, COL_TR, rowid, rowoff, outb)
            attend(stb, 2, stab_v, es_v, COL_ST, rowid, rowoff, outb)
            return carry

        lax.fori_loop(0, NGROUP, group_ac, 0)
        g_descs.pop(c).wait()

        def group_b(g, carry):
            rbase = g * 16
            for j in range(16):
                r = rbase + j
                roff = r * OUT_D
                for k in range(4):
                    outb[pl.ds(roff + k * 16, 16)] = rowsb[r, pl.ds(k * 16, 16)]
            return carry

        lax.fori_loop(0, NGROUP, group_b, 0)
        out_descs[c] = pltpu.async_copy(
            out_v.at[b, pl.ds(0, OUT_W)],
            out.at[pl.ds(base * OUT_D, OUT_W)], sem_out.at[b])
    for c in sorted(out_descs):
        out_descs.pop(c).wait()


def kernel(unit_type_ids, ability_indices, trait_indices, status_indices,
           numerical, resistances, defenses, movement_costs,
           unit_type_table, ability_table, trait_table, status_table,
           ability_query, trait_query, status_query):
    mesh = plsc.VectorSubcoreMesh(core_axis_name="c", subcore_axis_name="s")
    f32 = jnp.float32
    i32 = jnp.int32
    kfn = pl.kernel(
        _body,
        mesh=mesh,
        compiler_params=pltpu.CompilerParams(
            use_tc_tiling_on_sc=False, needs_layout_passes=False),
        out_type=jax.ShapeDtypeStruct((B * OUT_D,), f32),
        scratch_types=[
            pltpu.VMEM((2, CHUNK), i32),
            pltpu.VMEM((2, CHUNK, 4), i32),
            pltpu.VMEM((2, CHUNK, 3), i32),
            pltpu.VMEM((2, CHUNK, 2), i32),
            pltpu.VMEM((2, CHUNK * 11 + 16), f32),
            pltpu.VMEM((2, CHUNK * 6 + 16), f32),
            pltpu.VMEM((2, CHUNK * 10 + 16), f32),
            pltpu.VMEM((2, CHUNK * 10 + 16), f32),
            pltpu.VMEM((2, CHUNK, 64), f32),
            pltpu.VMEM((2, OUT_W + 16), f32),
            pltpu.VMEM((16, 16), f32),
            pltpu.VMEM((16, 16), f32),
            pltpu.VMEM((16, 16), f32),
            pltpu.VMEM((16,), f32),
            pltpu.VMEM((16,), f32),
            pltpu.VMEM((16,), f32),
            pltpu.VMEM((16,), f32),
            pltpu.VMEM((16,), f32),
            pltpu.VMEM((16,), f32),
            pltpu.SemaphoreType.DMA((2,)),
            pltpu.SemaphoreType.DMA((2,)),
            pltpu.SemaphoreType.DMA((2,)),
        ],
    )
    out_flat = kfn(
        unit_type_ids.astype(i32),
        ability_indices.astype(i32),
        trait_indices.astype(i32),
        status_indices.astype(i32),
        numerical.reshape(-1),
        resistances.reshape(-1),
        defenses.reshape(-1),
        movement_costs.reshape(-1),
        unit_type_table, ability_table, trait_table, status_table,
        ability_query, trait_query, status_query,
    )
    return out_flat.reshape(B, OUT_D)
